# baseline (speedup 1.0000x reference)
import functools

import jax
import jax.numpy as jnp
from jax import lax
from jax.experimental import pallas as pl
from jax.experimental.pallas import tpu as pltpu
from jax.experimental.pallas import tpu_sc as plsc

_B = 4096
_L = 50
_D = 128
_LANES = 16
_NCHUNK = _D // _LANES

_info = plsc.get_sparse_core_info()
_NC = _info.num_cores
_NS = _info.num_subcores
_NW = _NC * _NS
_BPW = _B // _NW
_RPS = 2
_IDXW = _RPS * _L
_STEPS = _BPW // _RPS
_IDXP = 104


def _sc_pool(table, idxT):
    mesh = plsc.VectorSubcoreMesh(core_axis_name="c", subcore_axis_name="s")

    @functools.partial(
        pl.kernel,
        mesh=mesh,
        out_type=jax.ShapeDtypeStruct((_B, _D), jnp.float32),
        scratch_types=[
            pltpu.VMEM((_L, _BPW), jnp.int32),
            pltpu.VMEM((_BPW, _D), jnp.float32),
            pltpu.VMEM((_BPW, _D), jnp.float32),
            pltpu.VMEM((_BPW, _D), jnp.float32),
            pltpu.VMEM((_BPW, _D), jnp.float32),
            pltpu.VMEM((_BPW, _D), jnp.float32),
            pltpu.SemaphoreType.DMA,
            pltpu.SemaphoreType.DMA,
            pltpu.SemaphoreType.DMA,
            pltpu.SemaphoreType.DMA,
        ],
    )
    def k(table_hbm, idx_hbm, out_hbm, idx_v, b0, b1, b2, b3, out_v,
          s0, s1, s2, s3):
        wid = lax.axis_index("s") * _NC + lax.axis_index("c")
        pltpu.sync_copy(idx_hbm.at[:, pl.ds(wid * _BPW, _BPW)], idx_v)
        bufs = (b0, b1, b2, b3)
        sems = (s0, s1, s2, s3)
        nbuf = 4

        zero = jnp.zeros((_LANES,), jnp.float32)

        def zbody(r, carry):
            for j in range(_NCHUNK):
                out_v[r, pl.ds(j * _LANES, _LANES)] = zero
            return carry

        lax.fori_loop(0, _BPW, zbody, 0, unroll=8)

        def start(t, p):
            pltpu.make_async_copy(
                table_hbm.at[idx_v.at[t]], bufs[p], sems[p]
            ).start()

        def wait(p):
            pltpu.make_async_copy(
                table_hbm.at[idx_v.at[0]], bufs[p], sems[p]
            ).wait()

        def accum(p):
            rows_v = bufs[p]

            def body(i, carry):
                for j in range(_NCHUNK):
                    plsc.addupdate(
                        out_v.at[i, pl.ds(j * _LANES, _LANES)],
                        rows_v[i, pl.ds(j * _LANES, _LANES)],
                    )
                return carry

            lax.fori_loop(0, _BPW, body, 0, unroll=8)

        for p in range(nbuf - 1):
            start(p, p)

        nrounds = _L // nbuf
        nrem = _L - nrounds * nbuf

        def rnd(i, carry):
            t0 = nbuf * i
            for p in range(nbuf):
                t = t0 + p
                nxt = t + nbuf - 1

                @pl.when(nxt < _L)
                def _():
                    start(nxt, (p + nbuf - 1) % nbuf)

                wait(p)
                accum(p)
            return carry

        lax.fori_loop(0, nrounds, rnd, 0)
        for p in range(nrem):
            wait(p)
            accum(p)
        pltpu.sync_copy(out_v, out_hbm.at[pl.ds(wid * _BPW, _BPW)])

    return k(table, idxT)


def _mlp(sums, lengths, W1, b1, W2, b2):
    grid = 1
    blk = _B // grid
    out_n = W2.shape[1]


    def body(s_ref, l_ref, w1_ref, b1_ref, w2t_ref, b2_ref, o_ref):
        rep = s_ref[...] / l_ref[...].astype(jnp.float32)
        h = jnp.dot(rep, w1_ref[...], preferred_element_type=jnp.float32)
        h = jnp.maximum(h + b1_ref[...], 0.0)
        o_ref[...] = (
            jax.lax.dot_general(
                w2t_ref[...], h, (((1,), (1,)), ((), ())),
                preferred_element_type=jnp.float32,
            )
            + b2_ref[...]
        )

    lat = W1.shape[1]
    outT = pl.pallas_call(
        body,
        grid=(grid,),
        in_specs=[
            pl.BlockSpec((blk, _D), lambda i: (i, 0)),
            pl.BlockSpec((blk, 1), lambda i: (i, 0)),
            pl.BlockSpec((_D, lat), lambda i: (0, 0)),
            pl.BlockSpec((1, lat), lambda i: (0, 0)),
            pl.BlockSpec((out_n, lat), lambda i: (0, 0)),
            pl.BlockSpec((out_n, 1), lambda i: (0, 0)),
        ],
        out_specs=pl.BlockSpec((out_n, blk), lambda i: (0, i)),
        out_shape=jax.ShapeDtypeStruct((out_n, _B), jnp.float32),
    )(sums, lengths.reshape(_B, 1), W1, b1.reshape(1, lat),
      jnp.transpose(W2), b2.reshape(out_n, 1))
    return jnp.transpose(outT)


def kernel(x, lengths, table, W1, b1, W2, b2):
    sums = _sc_pool(table, jnp.transpose(x.astype(jnp.int32)))
    return _mlp(sums, lengths, W1, b1, W2, b2)


# speedup vs baseline: 1.3923x; 1.3923x over previous
import functools

import jax
import jax.numpy as jnp
from jax import lax
from jax.experimental import pallas as pl
from jax.experimental.pallas import tpu as pltpu
from jax.experimental.pallas import tpu_sc as plsc

_B = 4096
_L = 50
_D = 128
_LANES = 16
_NCHUNK = _D // _LANES

_info = plsc.get_sparse_core_info()
_NC = _info.num_cores
_NS = _info.num_subcores
_NW = _NC * _NS
_BPW = _B // _NW
_RPS = 2
_IDXW = _RPS * _L
_STEPS = _BPW // _RPS


def _sc_pool(table, idx2d):
    mesh = plsc.VectorSubcoreMesh(core_axis_name="c", subcore_axis_name="s")

    @functools.partial(
        pl.kernel,
        mesh=mesh,
        out_type=jax.ShapeDtypeStruct((_B, _D), jnp.float32),
        scratch_types=[
            pltpu.VMEM((_BPW, _L), jnp.int32),
            pltpu.VMEM((_IDXW, _D), jnp.float32),
            pltpu.VMEM((_IDXW, _D), jnp.float32),
            pltpu.VMEM((_IDXW, _D), jnp.float32),
            pltpu.VMEM((_IDXW, _D), jnp.float32),
            pltpu.VMEM((_BPW, _D), jnp.float32),
            pltpu.SemaphoreType.DMA,
            pltpu.SemaphoreType.DMA,
            pltpu.SemaphoreType.DMA,
            pltpu.SemaphoreType.DMA,
        ],
    )
    def k(table_hbm, idx_hbm, out_hbm, idx_v, b0, b1, b2, b3, out_v,
          s0, s1, s2, s3):
        wid = lax.axis_index("s") * _NC + lax.axis_index("c")
        pltpu.sync_copy(idx_hbm.at[pl.ds(wid * _BPW, _BPW)], idx_v)
        bufs = (b0, b1, b2, b3)
        sems = (s0, s1, s2, s3)
        nbuf = 4

        def start(t, p):
            pltpu.make_async_copy(
                table_hbm.at[idx_v.at[2 * t]],
                bufs[p].at[pl.ds(0, _L)], sems[p]
            ).start()
            pltpu.make_async_copy(
                table_hbm.at[idx_v.at[2 * t + 1]],
                bufs[p].at[pl.ds(_L, _L)], sems[p]
            ).start()

        def wait(p):
            pltpu.make_async_copy(
                table_hbm.at[idx_v.at[0]], bufs[p].at[pl.ds(0, _L)], sems[p]
            ).wait()
            pltpu.make_async_copy(
                table_hbm.at[idx_v.at[0]], bufs[p].at[pl.ds(_L, _L)], sems[p]
            ).wait()

        def accum(t, p):
            rows_v = bufs[p]
            for r in range(_RPS):
                def body(kk, accs):
                    return tuple(
                        accs[j] + rows_v[r * _L + kk, pl.ds(j * _LANES, _LANES)]
                        for j in range(_NCHUNK)
                    )
                accs = lax.fori_loop(
                    0, _L, body,
                    tuple(jnp.zeros((_LANES,), jnp.float32) for _ in range(_NCHUNK)),
                    unroll=5,
                )
                for j in range(_NCHUNK):
                    out_v[t * _RPS + r, pl.ds(j * _LANES, _LANES)] = accs[j]

        for p in range(nbuf - 1):
            start(p, p)

        def quad(i, carry):
            t0 = nbuf * i
            for p in range(nbuf):
                t = t0 + p
                nxt = t + nbuf - 1

                @pl.when(nxt < _STEPS)
                def _():
                    start(nxt, (p + nbuf - 1) % nbuf)

                wait(p)
                accum(t, p)
            return carry

        lax.fori_loop(0, _STEPS // nbuf, quad, 0)
        pltpu.sync_copy(out_v, out_hbm.at[pl.ds(wid * _BPW, _BPW)])

    return k(table, idx2d)


def _mlp(sums, lengths, W1, b1, W2, b2):
    grid = 1
    blk = _B // grid
    out_n = W2.shape[1]

    def body(s_ref, l_ref, w1_ref, b1_ref, w2t_ref, b2_ref, o_ref):
        rep = s_ref[...] / l_ref[...].astype(jnp.float32)
        h = jnp.dot(rep, w1_ref[...], preferred_element_type=jnp.float32)
        h = jnp.maximum(h + b1_ref[...], 0.0)
        o_ref[...] = (
            jax.lax.dot_general(
                w2t_ref[...], h, (((1,), (1,)), ((), ())),
                preferred_element_type=jnp.float32,
            )
            + b2_ref[...]
        )

    lat = W1.shape[1]
    outT = pl.pallas_call(
        body,
        grid=(grid,),
        in_specs=[
            pl.BlockSpec((blk, _D), lambda i: (i, 0)),
            pl.BlockSpec((blk, 1), lambda i: (i, 0)),
            pl.BlockSpec((_D, lat), lambda i: (0, 0)),
            pl.BlockSpec((1, lat), lambda i: (0, 0)),
            pl.BlockSpec((out_n, lat), lambda i: (0, 0)),
            pl.BlockSpec((out_n, 1), lambda i: (0, 0)),
        ],
        out_specs=pl.BlockSpec((out_n, blk), lambda i: (0, i)),
        out_shape=jax.ShapeDtypeStruct((out_n, _B), jnp.float32),
    )(sums, lengths.reshape(_B, 1), W1, b1.reshape(1, lat),
      jnp.transpose(W2), b2.reshape(out_n, 1))
    return jnp.transpose(outT)


def kernel(x, lengths, table, W1, b1, W2, b2):
    sums = _sc_pool(table, x.astype(jnp.int32))
    return _mlp(sums, lengths, W1, b1, W2, b2)


# speedup vs baseline: 1.4078x; 1.0112x over previous
import functools

import jax
import jax.numpy as jnp
from jax import lax
from jax.experimental import pallas as pl
from jax.experimental.pallas import tpu as pltpu
from jax.experimental.pallas import tpu_sc as plsc

_B = 4096
_L = 50
_D = 128
_LANES = 16
_NCHUNK = _D // _LANES

_info = plsc.get_sparse_core_info()
_NC = _info.num_cores
_NS = _info.num_subcores
_NW = _NC * _NS
_BPW = _B // _NW
_RPS = 2
_IDXW = _RPS * _L
_STEPS = _BPW // _RPS


def _sc_pool(table, idx2d):
    mesh = plsc.VectorSubcoreMesh(core_axis_name="c", subcore_axis_name="s")

    @functools.partial(
        pl.kernel,
        mesh=mesh,
        out_type=jax.ShapeDtypeStruct((_B, _D), jnp.float32),
        scratch_types=[
            pltpu.VMEM((_BPW, _L), jnp.int32),
            pltpu.VMEM((_IDXW, _D), jnp.float32),
            pltpu.VMEM((_IDXW, _D), jnp.float32),
            pltpu.VMEM((_IDXW, _D), jnp.float32),
            pltpu.VMEM((_IDXW, _D), jnp.float32),
            pltpu.VMEM((_BPW, _D), jnp.float32),
            pltpu.SemaphoreType.DMA,
            pltpu.SemaphoreType.DMA,
            pltpu.SemaphoreType.DMA,
            pltpu.SemaphoreType.DMA,
        ],
    )
    def k(table_hbm, idx_hbm, out_hbm, idx_v, b0, b1, b2, b3, out_v,
          s0, s1, s2, s3):
        wid = lax.axis_index("s") * _NC + lax.axis_index("c")
        pltpu.sync_copy(idx_hbm.at[pl.ds(wid * _BPW, _BPW)], idx_v)
        bufs = (b0, b1, b2, b3)
        sems = (s0, s1, s2, s3)
        nbuf = 4

        def start(t, p):
            pltpu.make_async_copy(
                table_hbm.at[idx_v.at[2 * t]],
                bufs[p].at[pl.ds(0, _L)], sems[p]
            ).start()
            pltpu.make_async_copy(
                table_hbm.at[idx_v.at[2 * t + 1]],
                bufs[p].at[pl.ds(_L, _L)], sems[p]
            ).start()

        def wait(p):
            pltpu.make_async_copy(
                table_hbm.at[idx_v.at[0]], bufs[p].at[pl.ds(0, _L)], sems[p]
            ).wait()
            pltpu.make_async_copy(
                table_hbm.at[idx_v.at[0]], bufs[p].at[pl.ds(_L, _L)], sems[p]
            ).wait()

        def accum(t, p):
            rows_v = bufs[p]
            for r in range(_RPS):
                def body(kk, accs):
                    return tuple(
                        accs[j] + rows_v[r * _L + kk, pl.ds(j * _LANES, _LANES)]
                        for j in range(_NCHUNK)
                    )
                accs = lax.fori_loop(
                    0, _L, body,
                    tuple(jnp.zeros((_LANES,), jnp.float32) for _ in range(_NCHUNK)),
                    unroll=2,
                )
                for j in range(_NCHUNK):
                    out_v[t * _RPS + r, pl.ds(j * _LANES, _LANES)] = accs[j]

        for p in range(nbuf - 1):
            start(p, p)

        def quad(i, carry):
            t0 = nbuf * i
            for p in range(nbuf):
                t = t0 + p
                nxt = t + nbuf - 1

                @pl.when(nxt < _STEPS)
                def _():
                    start(nxt, (p + nbuf - 1) % nbuf)

                wait(p)
                accum(t, p)
            return carry

        lax.fori_loop(0, _STEPS // nbuf, quad, 0)
        pltpu.sync_copy(out_v, out_hbm.at[pl.ds(wid * _BPW, _BPW)])

    return k(table, idx2d)


def _mlp(sums, lengths, W1, b1, W2, b2):
    grid = 1
    blk = _B // grid
    out_n = W2.shape[1]

    def body(s_ref, l_ref, w1_ref, b1_ref, w2t_ref, b2_ref, o_ref):
        rep = s_ref[...] / l_ref[...].astype(jnp.float32)
        h = jnp.dot(rep, w1_ref[...], preferred_element_type=jnp.float32)
        h = jnp.maximum(h + b1_ref[...], 0.0)
        o_ref[...] = (
            jax.lax.dot_general(
                w2t_ref[...], h, (((1,), (1,)), ((), ())),
                preferred_element_type=jnp.float32,
            )
            + b2_ref[...]
        )

    lat = W1.shape[1]
    outT = pl.pallas_call(
        body,
        grid=(grid,),
        in_specs=[
            pl.BlockSpec((blk, _D), lambda i: (i, 0)),
            pl.BlockSpec((blk, 1), lambda i: (i, 0)),
            pl.BlockSpec((_D, lat), lambda i: (0, 0)),
            pl.BlockSpec((1, lat), lambda i: (0, 0)),
            pl.BlockSpec((out_n, lat), lambda i: (0, 0)),
            pl.BlockSpec((out_n, 1), lambda i: (0, 0)),
        ],
        out_specs=pl.BlockSpec((out_n, blk), lambda i: (0, i)),
        out_shape=jax.ShapeDtypeStruct((out_n, _B), jnp.float32),
    )(sums, lengths.reshape(_B, 1), W1, b1.reshape(1, lat),
      jnp.transpose(W2), b2.reshape(out_n, 1))
    return jnp.transpose(outT)


def kernel(x, lengths, table, W1, b1, W2, b2):
    sums = _sc_pool(table, x.astype(jnp.int32))
    return _mlp(sums, lengths, W1, b1, W2, b2)


# speedup vs baseline: 1.4154x; 1.0054x over previous
import functools

import jax
import jax.numpy as jnp
from jax import lax
from jax.experimental import pallas as pl
from jax.experimental.pallas import tpu as pltpu
from jax.experimental.pallas import tpu_sc as plsc

_B = 4096
_L = 50
_D = 128
_LANES = 16
_NCHUNK = _D // _LANES

_info = plsc.get_sparse_core_info()
_NC = _info.num_cores
_NS = _info.num_subcores
_NW = _NC * _NS
_BPW = _B // _NW
_RPS = 2
_IDXW = _RPS * _L
_STEPS = _BPW // _RPS


def _sc_pool(table, idx2d):
    mesh = plsc.VectorSubcoreMesh(core_axis_name="c", subcore_axis_name="s")

    @functools.partial(
        pl.kernel,
        mesh=mesh,
        out_type=jax.ShapeDtypeStruct((_B, _D), jnp.float32),
        scratch_types=[
            pltpu.VMEM((_BPW, _L), jnp.int32),
            pltpu.VMEM((_IDXW, _D), jnp.float32),
            pltpu.VMEM((_IDXW, _D), jnp.float32),
            pltpu.VMEM((_IDXW, _D), jnp.float32),
            pltpu.VMEM((_IDXW, _D), jnp.float32),
            pltpu.VMEM((_BPW, _D), jnp.float32),
            pltpu.SemaphoreType.DMA,
            pltpu.SemaphoreType.DMA,
            pltpu.SemaphoreType.DMA,
            pltpu.SemaphoreType.DMA,
        ],
    )
    def k(table_hbm, idx_hbm, out_hbm, idx_v, b0, b1, b2, b3, out_v,
          s0, s1, s2, s3):
        wid = lax.axis_index("s") * _NC + lax.axis_index("c")
        pltpu.sync_copy(idx_hbm.at[pl.ds(wid * _BPW, _BPW)], idx_v)
        bufs = (b0, b1, b2, b3)
        sems = (s0, s1, s2, s3)
        nbuf = 4

        def start(t, p):
            pltpu.make_async_copy(
                table_hbm.at[idx_v.at[2 * t]],
                bufs[p].at[pl.ds(0, _L)], sems[p]
            ).start()
            pltpu.make_async_copy(
                table_hbm.at[idx_v.at[2 * t + 1]],
                bufs[p].at[pl.ds(_L, _L)], sems[p]
            ).start()

        def wait(p):
            pltpu.make_async_copy(
                table_hbm.at[idx_v.at[0]], bufs[p].at[pl.ds(0, _L)], sems[p]
            ).wait()
            pltpu.make_async_copy(
                table_hbm.at[idx_v.at[0]], bufs[p].at[pl.ds(_L, _L)], sems[p]
            ).wait()

        def accum(t, p):
            rows_v = bufs[p]
            for r in range(_RPS):
                def body(kk, accs):
                    return tuple(
                        accs[j] + rows_v[r * _L + kk, pl.ds(j * _LANES, _LANES)]
                        for j in range(_NCHUNK)
                    )
                accs = lax.fori_loop(
                    0, _L, body,
                    tuple(jnp.zeros((_LANES,), jnp.float32) for _ in range(_NCHUNK)),
                    unroll=1,
                )
                for j in range(_NCHUNK):
                    out_v[t * _RPS + r, pl.ds(j * _LANES, _LANES)] = accs[j]

        for p in range(nbuf - 1):
            start(p, p)

        def quad(i, carry):
            t0 = nbuf * i
            for p in range(nbuf):
                t = t0 + p
                nxt = t + nbuf - 1

                @pl.when(nxt < _STEPS)
                def _():
                    start(nxt, (p + nbuf - 1) % nbuf)

                wait(p)
                accum(t, p)
            return carry

        lax.fori_loop(0, _STEPS // nbuf, quad, 0)
        pltpu.sync_copy(out_v, out_hbm.at[pl.ds(wid * _BPW, _BPW)])

    return k(table, idx2d)


def _mlp(sums, lengths, W1, b1, W2, b2):
    grid = 1
    blk = _B // grid
    out_n = W2.shape[1]

    def body(s_ref, l_ref, w1_ref, b1_ref, w2t_ref, b2_ref, o_ref):
        rep = s_ref[...] / l_ref[...].astype(jnp.float32)
        h = jnp.dot(rep, w1_ref[...], preferred_element_type=jnp.float32)
        h = jnp.maximum(h + b1_ref[...], 0.0)
        o_ref[...] = (
            jax.lax.dot_general(
                w2t_ref[...], h, (((1,), (1,)), ((), ())),
                preferred_element_type=jnp.float32,
            )
            + b2_ref[...]
        )

    lat = W1.shape[1]
    outT = pl.pallas_call(
        body,
        grid=(grid,),
        in_specs=[
            pl.BlockSpec((blk, _D), lambda i: (i, 0)),
            pl.BlockSpec((blk, 1), lambda i: (i, 0)),
            pl.BlockSpec((_D, lat), lambda i: (0, 0)),
            pl.BlockSpec((1, lat), lambda i: (0, 0)),
            pl.BlockSpec((out_n, lat), lambda i: (0, 0)),
            pl.BlockSpec((out_n, 1), lambda i: (0, 0)),
        ],
        out_specs=pl.BlockSpec((out_n, blk), lambda i: (0, i)),
        out_shape=jax.ShapeDtypeStruct((out_n, _B), jnp.float32),
    )(sums, lengths.reshape(_B, 1), W1, b1.reshape(1, lat),
      jnp.transpose(W2), b2.reshape(out_n, 1))
    return jnp.transpose(outT)


def kernel(x, lengths, table, W1, b1, W2, b2):
    sums = _sc_pool(table, x.astype(jnp.int32))
    return _mlp(sums, lengths, W1, b1, W2, b2)
